# Initial kernel scaffold; baseline (speedup 1.0000x reference)
import jax
import jax.numpy as jnp
from jax.experimental import pallas as pl


def kernel(input_features, adj, weight, bias):
    raise NotImplementedError("write your pallas kernel here")


# speedup vs baseline: 1.0375x; 1.0375x over previous
import jax
import jax.numpy as jnp
from jax.experimental import pallas as pl
from jax.experimental.pallas import tpu as pltpu


def _pick_bm(n: int, cap: int = 512) -> int:
    best = 8
    for bm in range(8, cap + 1, 8):
        if n % bm == 0:
            best = bm
    return best


def _gcn_block(adj_ref, x_ref, w_ref, b_ref, out_ref):
    a = adj_ref[...].astype(jnp.bfloat16)
    x = x_ref[...].astype(jnp.bfloat16)
    t = jnp.dot(a, x, preferred_element_type=jnp.float32)
    out_ref[...] = (
        jnp.dot(t, w_ref[...], preferred_element_type=jnp.float32) + b_ref[...]
    )


def kernel(input_features, adj, weight, bias):
    n, d_in = input_features.shape
    d_out = weight.shape[1]
    bm = _pick_bm(n)
    grid = (n // bm,)
    bias2d = bias.reshape(1, d_out)
    out = pl.pallas_call(
        _gcn_block,
        grid=grid,
        in_specs=[
            pl.BlockSpec((bm, n), lambda i: (i, 0)),
            pl.BlockSpec((n, d_in), lambda i: (0, 0)),
            pl.BlockSpec((d_in, d_out), lambda i: (0, 0)),
            pl.BlockSpec((1, d_out), lambda i: (0, 0)),
        ],
        out_specs=pl.BlockSpec((bm, d_out), lambda i: (i, 0)),
        out_shape=jax.ShapeDtypeStruct((n, d_out), jnp.float32),
        compiler_params=pltpu.CompilerParams(
            dimension_semantics=("arbitrary",),
        ),
    )(adj, input_features, weight, bias2d)
    return out
